# Initial kernel scaffold; baseline (speedup 1.0000x reference)
import jax
import jax.numpy as jnp
from jax.experimental import pallas as pl


def kernel(weight, indices, offsets):
    raise NotImplementedError("write your pallas kernel here")


# speedup vs baseline: 42.7122x; 42.7122x over previous
import functools

import jax
import jax.numpy as jnp
from jax import lax
from jax.experimental import pallas as pl
from jax.experimental.pallas import tpu as pltpu
from jax.experimental.pallas import tpu_sc as plsc

VOCAB = 1000000
D = 128
N_IDX = 104217

NC = 2
NS = 16
NW = NC * NS

CHUNK = 128
NCHUNKS = 26
B_PER_W = CHUNK * NCHUNKS
B_PAD = B_PER_W * NW


def _gather_body(table_hbm, idx_hbm, out_hbm, idx_v, rows_v, sem):
    wid = lax.axis_index("s") * NC + lax.axis_index("c")
    base = wid * B_PER_W
    pltpu.sync_copy(idx_hbm.at[wid], idx_v)

    def chunk(i, carry):
        pltpu.async_copy(table_hbm.at[idx_v.at[i]], rows_v, sem).wait()
        pltpu.sync_copy(rows_v, out_hbm.at[pl.ds(base + i * CHUNK, CHUNK)])
        return carry

    lax.fori_loop(0, NCHUNKS, chunk, 0)


@jax.jit
def _gather(weight, idx3):
    mesh = plsc.VectorSubcoreMesh(core_axis_name="c", subcore_axis_name="s")
    f = pl.kernel(
        _gather_body,
        mesh=mesh,
        out_type=jax.ShapeDtypeStruct((B_PAD, D), jnp.float32),
        scratch_types=[
            pltpu.VMEM((NCHUNKS, CHUNK), jnp.int32),
            pltpu.VMEM((CHUNK, D), jnp.float32),
            pltpu.SemaphoreType.DMA,
        ],
    )
    return f(weight, idx3)


def kernel(weight, indices, offsets):
    idx = indices.astype(jnp.int32)
    idx = jnp.pad(idx, (0, B_PAD - N_IDX))
    idx3 = idx.reshape(NW, NCHUNKS, CHUNK)
    out = _gather(weight, idx3)
    return out[:N_IDX]


# speedup vs baseline: 46.9533x; 1.0993x over previous
import functools

import jax
import jax.numpy as jnp
from jax import lax
from jax.experimental import pallas as pl
from jax.experimental.pallas import tpu as pltpu
from jax.experimental.pallas import tpu_sc as plsc

VOCAB = 1000000
D = 128
N_IDX = 104217

NC = 2
NS = 16
NW = NC * NS

CHUNK = 128
NCHUNKS = 26
B_PER_W = CHUNK * NCHUNKS
B_PAD = B_PER_W * NW


def _gather_body(table_hbm, idx_hbm, out_hbm, idx_v, rows0, rows1, sem0, sem1):
    wid = lax.axis_index("s") * NC + lax.axis_index("c")
    base = wid * B_PER_W
    pltpu.sync_copy(idx_hbm.at[wid], idx_v)

    bufs = (rows0, rows1)
    sems = (sem0, sem1)
    pltpu.async_copy(table_hbm.at[idx_v.at[0]], rows0, sem0)

    def group(g, carry):
        for b in (0, 1):
            i = g * 2 + b
            @pl.when(i + 1 < NCHUNKS)
            def _():
                pltpu.async_copy(
                    table_hbm.at[idx_v.at[i + 1]], bufs[1 - b], sems[1 - b]
                )

            pltpu.make_async_copy(table_hbm.at[idx_v.at[i]], bufs[b], sems[b]).wait()
            pltpu.sync_copy(bufs[b], out_hbm.at[pl.ds(base + i * CHUNK, CHUNK)])
        return carry

    lax.fori_loop(0, NCHUNKS // 2, group, 0)


@jax.jit
def _gather(weight, idx3):
    mesh = plsc.VectorSubcoreMesh(core_axis_name="c", subcore_axis_name="s")
    f = pl.kernel(
        _gather_body,
        mesh=mesh,
        out_type=jax.ShapeDtypeStruct((B_PAD, D), jnp.float32),
        scratch_types=[
            pltpu.VMEM((NCHUNKS, CHUNK), jnp.int32),
            pltpu.VMEM((CHUNK, D), jnp.float32),
            pltpu.VMEM((CHUNK, D), jnp.float32),
            pltpu.SemaphoreType.DMA,
            pltpu.SemaphoreType.DMA,
        ],
    )
    return f(weight, idx3)


def kernel(weight, indices, offsets):
    idx = indices.astype(jnp.int32)
    idx = jnp.pad(idx, (0, B_PAD - N_IDX))
    idx3 = idx.reshape(NW, NCHUNKS, CHUNK)
    out = _gather(weight, idx3)
    return out[:N_IDX]
